# Initial kernel scaffold; baseline (speedup 1.0000x reference)
import jax
import jax.numpy as jnp
from jax.experimental import pallas as pl


def kernel(z, nuclare_table, elec_W, ls_W, ls_b):
    raise NotImplementedError("write your pallas kernel here")


# speedup vs baseline: 1.5916x; 1.5916x over previous
import functools

import numpy as np
import jax
import jax.numpy as jnp
from jax import lax
from jax.experimental import pallas as pl
from jax.experimental.pallas import tpu as pltpu
from jax.experimental.pallas import tpu_sc as plsc

_ELEC_ROWS = np.array(
    [
        [0, 0, 0, 0, 0, 0, 0, 0, 0, 0, 0, 0, 0, 0, 0, 0],
        [0, 1, 0, 0, 0, 0, 0, 0, 0, 0, 0, 0, 0, 0, 0, 0],
        [2, 0, 0, 0, 0, 0, 0, 0, 0, 0, 0, 0, 0, 0, 0, 0],
        [2, 0, 0, 1, 0, 0, 0, 0, 0, 0, 0, 0, 0, 0, 0, 0],
        [2, 0, 2, 0, 0, 0, 0, 0, 0, 0, 0, 0, 0, 0, 0, 0],
        [2, 0, 2, 0, 0, 1, 0, 0, 0, 0, 0, 0, 0, 0, 0, 0],
        [2, 0, 2, 0, 0, 2, 0, 0, 0, 0, 0, 0, 0, 0, 0, 0],
        [2, 0, 2, 0, 0, 3, 0, 0, 0, 0, 0, 0, 0, 0, 0, 0],
        [2, 0, 2, 0, 2, 2, 0, 0, 0, 0, 0, 0, 0, 0, 0, 0],
        [2, 0, 2, 0, 4, 1, 0, 0, 0, 0, 0, 0, 0, 0, 0, 0],
    ],
    dtype=np.float32,
)
_ELEC_NORM = _ELEC_ROWS / _ELEC_ROWS.max()
_VPAD = 16
_ELEC_PAD = np.zeros((_VPAD, 16), dtype=np.float32)
_ELEC_PAD[:10] = _ELEC_NORM


def _fused_table_body(elec_ref, nuc_ref, elec_w_ref, ls_w_ref, ls_b_ref, out_ref):
    elec_emb = jnp.dot(elec_ref[...], elec_w_ref[...],
                       preferred_element_type=jnp.float32)
    h = nuc_ref[...] + elec_emb
    h = jnp.dot(h, ls_w_ref[...], preferred_element_type=jnp.float32)
    h = h + ls_b_ref[...]
    out_ref[...] = h * jax.nn.sigmoid(h)


def _compute_fused_table(nuclare_table, elec_W, ls_W, ls_b):
    vocab, num_features = nuclare_table.shape
    nuc_pad = jnp.zeros((_VPAD, num_features), jnp.float32).at[:vocab].set(nuclare_table)
    elec_pad = jnp.asarray(_ELEC_PAD)
    return pl.pallas_call(
        _fused_table_body,
        out_shape=jax.ShapeDtypeStruct((_VPAD, num_features), jnp.float32),
    )(elec_pad, nuc_pad, elec_W, ls_W, ls_b.reshape(1, num_features))


def _make_sc_gather(B, D, nc, ns, chunk=128):
    nw = nc * ns
    b_per_w = B // nw
    n_chunks = b_per_w // chunk
    mesh = plsc.VectorSubcoreMesh(core_axis_name="c", subcore_axis_name="s")

    @functools.partial(
        pl.kernel,
        mesh=mesh,
        out_type=jax.ShapeDtypeStruct((B, D), jnp.float32),
        scratch_types=[
            pltpu.VMEM((chunk,), jnp.int32),
            pltpu.VMEM((chunk, D), jnp.float32),
            pltpu.SemaphoreType.DMA,
        ],
    )
    def gather_kernel(z_hbm, table_hbm, out_hbm, idx_v, rows_v, sem):
        wid = lax.axis_index("s") * nc + lax.axis_index("c")
        base0 = wid * b_per_w

        def body(i, carry):
            base = pl.multiple_of(base0 + i * chunk, 8)
            pltpu.sync_copy(z_hbm.at[pl.ds(base, chunk)], idx_v)
            pltpu.async_copy(table_hbm.at[idx_v], rows_v, sem).wait()
            pltpu.sync_copy(rows_v, out_hbm.at[pl.ds(base, chunk)])
            return carry

        lax.fori_loop(0, n_chunks, body, 0)

    return gather_kernel


def kernel(z, nuclare_table, elec_W, ls_W, ls_b):
    Bz, L = z.shape
    num_features = nuclare_table.shape[1]
    B = Bz * L

    fused = _compute_fused_table(nuclare_table, elec_W, ls_W, ls_b)

    info = plsc.get_sparse_core_info()
    gather_kernel = _make_sc_gather(B, num_features, info.num_cores,
                                    info.num_subcores)
    z_flat = z.reshape(B).astype(jnp.int32)
    out = gather_kernel(z_flat, fused)
    return out.reshape(Bz, L, num_features)


# speedup vs baseline: 1.6059x; 1.0090x over previous
import functools

import numpy as np
import jax
import jax.numpy as jnp
from jax import lax
from jax.experimental import pallas as pl
from jax.experimental.pallas import tpu as pltpu
from jax.experimental.pallas import tpu_sc as plsc

_ELEC_ROWS = np.array(
    [
        [0, 0, 0, 0, 0, 0, 0, 0, 0, 0, 0, 0, 0, 0, 0, 0],
        [0, 1, 0, 0, 0, 0, 0, 0, 0, 0, 0, 0, 0, 0, 0, 0],
        [2, 0, 0, 0, 0, 0, 0, 0, 0, 0, 0, 0, 0, 0, 0, 0],
        [2, 0, 0, 1, 0, 0, 0, 0, 0, 0, 0, 0, 0, 0, 0, 0],
        [2, 0, 2, 0, 0, 0, 0, 0, 0, 0, 0, 0, 0, 0, 0, 0],
        [2, 0, 2, 0, 0, 1, 0, 0, 0, 0, 0, 0, 0, 0, 0, 0],
        [2, 0, 2, 0, 0, 2, 0, 0, 0, 0, 0, 0, 0, 0, 0, 0],
        [2, 0, 2, 0, 0, 3, 0, 0, 0, 0, 0, 0, 0, 0, 0, 0],
        [2, 0, 2, 0, 2, 2, 0, 0, 0, 0, 0, 0, 0, 0, 0, 0],
        [2, 0, 2, 0, 4, 1, 0, 0, 0, 0, 0, 0, 0, 0, 0, 0],
    ],
    dtype=np.float32,
)
_ELEC_NORM = _ELEC_ROWS / _ELEC_ROWS.max()
_VPAD = 16
_ELEC_PAD = np.zeros((_VPAD, 16), dtype=np.float32)
_ELEC_PAD[:10] = _ELEC_NORM


def _fused_table_body(elec_ref, nuc_ref, elec_w_ref, ls_w_ref, ls_b_ref, out_ref):
    elec_emb = jnp.dot(elec_ref[...], elec_w_ref[...],
                       preferred_element_type=jnp.float32)
    h = nuc_ref[...] + elec_emb
    h = jnp.dot(h, ls_w_ref[...], preferred_element_type=jnp.float32)
    h = h + ls_b_ref[...]
    out_ref[...] = h * jax.nn.sigmoid(h)


def _compute_fused_table(nuclare_table, elec_W, ls_W, ls_b):
    vocab, num_features = nuclare_table.shape
    nuc_pad = jnp.zeros((_VPAD, num_features), jnp.float32).at[:vocab].set(nuclare_table)
    elec_pad = jnp.asarray(_ELEC_PAD)
    return pl.pallas_call(
        _fused_table_body,
        out_shape=jax.ShapeDtypeStruct((_VPAD, num_features), jnp.float32),
    )(elec_pad, nuc_pad, elec_W, ls_W, ls_b.reshape(1, num_features))


_NBUF = 4


def _make_sc_gather(B, D, nc, ns, chunk=128):
    nw = nc * ns
    b_per_w = B // nw
    n_chunks = b_per_w // chunk
    n_groups = n_chunks // _NBUF
    assert n_chunks % _NBUF == 0
    mesh = plsc.VectorSubcoreMesh(core_axis_name="c", subcore_axis_name="s")

    @functools.partial(
        pl.kernel,
        mesh=mesh,
        out_type=jax.ShapeDtypeStruct((B, D), jnp.float32),
        scratch_types=(
            [pltpu.VMEM((n_chunks, chunk), jnp.int32)]
            + [pltpu.VMEM((chunk, D), jnp.float32) for _ in range(_NBUF)]
            + [pltpu.SemaphoreType.DMA for _ in range(2 * _NBUF)]
        ),
    )
    def gather_kernel(z_hbm, table_hbm, out_hbm, idx_all, *bufs_and_sems):
        rows = bufs_and_sems[:_NBUF]
        gsems = bufs_and_sems[_NBUF:2 * _NBUF]
        wsems = bufs_and_sems[2 * _NBUF:]
        wid = lax.axis_index("s") * nc + lax.axis_index("c")
        chunk0 = wid * n_chunks

        pltpu.sync_copy(z_hbm.at[pl.ds(chunk0, n_chunks)], idx_all)

        def gather_wait(b):
            pltpu.make_async_copy(
                table_hbm.at[idx_all.at[0]], rows[b], gsems[b]).wait()

        def write_wait(b):
            pltpu.make_async_copy(
                rows[b], out_hbm.at[pl.ds(0, chunk)], wsems[b]).wait()

        for b in range(_NBUF):
            pltpu.async_copy(table_hbm.at[idx_all.at[b]], rows[b], gsems[b])

        def group(g, carry):
            j0 = g * _NBUF
            for b in range(_NBUF):
                gather_wait(b)
                pltpu.async_copy(
                    rows[b],
                    out_hbm.at[pl.ds((chunk0 + j0 + b) * chunk, chunk)],
                    wsems[b])
            for b in range(_NBUF):
                jn = j0 + b + _NBUF

                @pl.when(jn < n_chunks)
                def _():
                    write_wait(b)
                    pltpu.async_copy(
                        table_hbm.at[idx_all.at[jn]], rows[b], gsems[b])

            return carry

        lax.fori_loop(0, n_groups, group, 0)

        for b in range(_NBUF):
            write_wait(b)

    return gather_kernel


def kernel(z, nuclare_table, elec_W, ls_W, ls_b):
    Bz, L = z.shape
    num_features = nuclare_table.shape[1]
    B = Bz * L

    fused = _compute_fused_table(nuclare_table, elec_W, ls_W, ls_b)

    info = plsc.get_sparse_core_info()
    gather_kernel = _make_sc_gather(B, num_features, info.num_cores,
                                    info.num_subcores)
    z2d = z.reshape(B // 128, 128).astype(jnp.int32)
    out = gather_kernel(z2d, fused)
    return out.reshape(Bz, L, num_features)


# speedup vs baseline: 8.2979x; 5.1670x over previous
import functools

import numpy as np
import jax
import jax.numpy as jnp
from jax import lax
from jax.experimental import pallas as pl
from jax.experimental.pallas import tpu as pltpu
from jax.experimental.pallas import tpu_sc as plsc

_ELEC_ROWS = np.array(
    [
        [0, 0, 0, 0, 0, 0, 0, 0, 0, 0, 0, 0, 0, 0, 0, 0],
        [0, 1, 0, 0, 0, 0, 0, 0, 0, 0, 0, 0, 0, 0, 0, 0],
        [2, 0, 0, 0, 0, 0, 0, 0, 0, 0, 0, 0, 0, 0, 0, 0],
        [2, 0, 0, 1, 0, 0, 0, 0, 0, 0, 0, 0, 0, 0, 0, 0],
        [2, 0, 2, 0, 0, 0, 0, 0, 0, 0, 0, 0, 0, 0, 0, 0],
        [2, 0, 2, 0, 0, 1, 0, 0, 0, 0, 0, 0, 0, 0, 0, 0],
        [2, 0, 2, 0, 0, 2, 0, 0, 0, 0, 0, 0, 0, 0, 0, 0],
        [2, 0, 2, 0, 0, 3, 0, 0, 0, 0, 0, 0, 0, 0, 0, 0],
        [2, 0, 2, 0, 2, 2, 0, 0, 0, 0, 0, 0, 0, 0, 0, 0],
        [2, 0, 2, 0, 4, 1, 0, 0, 0, 0, 0, 0, 0, 0, 0, 0],
    ],
    dtype=np.float32,
)
_ELEC_NORM = _ELEC_ROWS / _ELEC_ROWS.max()
_VPAD = 16
_ELEC_PAD = np.zeros((_VPAD, 16), dtype=np.float32)
_ELEC_PAD[:10] = _ELEC_NORM


def _fused_table_body(elec_ref, nuc_ref, elec_w_ref, ls_w_ref, ls_b_ref, out_ref):
    elec_emb = jnp.dot(elec_ref[...], elec_w_ref[...],
                       preferred_element_type=jnp.float32)
    h = nuc_ref[...] + elec_emb
    h = jnp.dot(h, ls_w_ref[...], preferred_element_type=jnp.float32)
    h = h + ls_b_ref[...]
    out_ref[...] = h * jax.nn.sigmoid(h)


def _compute_fused_table(nuclare_table, elec_W, ls_W, ls_b):
    vocab, num_features = nuclare_table.shape
    nuc_pad = jnp.zeros((_VPAD, num_features), jnp.float32).at[:vocab].set(nuclare_table)
    elec_pad = jnp.asarray(_ELEC_PAD)
    return pl.pallas_call(
        _fused_table_body,
        out_shape=jax.ShapeDtypeStruct((_VPAD, num_features), jnp.float32),
    )(elec_pad, nuc_pad, elec_W, ls_W, ls_b.reshape(1, num_features))


_NBUF = 4


def _make_sc_gather(B, D, nc, ns, chunk=128):
    nw = nc * ns
    b_per_w = B // nw
    n_chunks = b_per_w // chunk
    n_groups = n_chunks // _NBUF
    assert n_chunks % _NBUF == 0
    mesh = plsc.VectorSubcoreMesh(core_axis_name="c", subcore_axis_name="s")

    @functools.partial(
        pl.kernel,
        mesh=mesh,
        out_type=jax.ShapeDtypeStruct((B, D), jnp.float32),
        scratch_types=(
            [pltpu.VMEM((n_chunks, chunk), jnp.int32)]
            + [pltpu.VMEM((chunk, D), jnp.float32) for _ in range(_NBUF)]
            + [pltpu.SemaphoreType.DMA for _ in range(2 * _NBUF)]
        ),
    )
    def gather_kernel(z_hbm, table_hbm, out_hbm, idx_all, *bufs_and_sems):
        rows = bufs_and_sems[:_NBUF]
        gsems = bufs_and_sems[_NBUF:2 * _NBUF]
        wsems = bufs_and_sems[2 * _NBUF:]
        wid = lax.axis_index("s") * nc + lax.axis_index("c")
        chunk0 = wid * n_chunks
        tbl = table_hbm.at[wid]

        pltpu.sync_copy(z_hbm.at[pl.ds(chunk0, n_chunks)], idx_all)

        def gather_wait(b):
            pltpu.make_async_copy(
                tbl.at[idx_all.at[0]], rows[b], gsems[b]).wait()

        def write_wait(b):
            pltpu.make_async_copy(
                rows[b], out_hbm.at[pl.ds(0, chunk)], wsems[b]).wait()

        for b in range(_NBUF):
            pltpu.async_copy(tbl.at[idx_all.at[b]], rows[b], gsems[b])

        def group(g, carry):
            j0 = g * _NBUF
            for b in range(_NBUF):
                gather_wait(b)
                pltpu.async_copy(
                    rows[b],
                    out_hbm.at[pl.ds((chunk0 + j0 + b) * chunk, chunk)],
                    wsems[b])
            for b in range(_NBUF):
                jn = j0 + b + _NBUF

                @pl.when(jn < n_chunks)
                def _():
                    write_wait(b)
                    pltpu.async_copy(tbl.at[idx_all.at[jn]], rows[b], gsems[b])

            return carry

        lax.fori_loop(0, n_groups, group, 0)

        for b in range(_NBUF):
            write_wait(b)

    return gather_kernel


def kernel(z, nuclare_table, elec_W, ls_W, ls_b):
    Bz, L = z.shape
    num_features = nuclare_table.shape[1]
    B = Bz * L

    fused = _compute_fused_table(nuclare_table, elec_W, ls_W, ls_b)

    info = plsc.get_sparse_core_info()
    nw = info.num_cores * info.num_subcores
    gather_kernel = _make_sc_gather(B, num_features, info.num_cores,
                                    info.num_subcores)
    z2d = z.reshape(B // 128, 128).astype(jnp.int32)
    table_rep = jnp.broadcast_to(fused[None], (nw, _VPAD, num_features))
    out = gather_kernel(z2d, table_rep)
    return out.reshape(Bz, L, num_features)


# speedup vs baseline: 8.3231x; 1.0030x over previous
import functools

import numpy as np
import jax
import jax.numpy as jnp
from jax import lax
from jax.experimental import pallas as pl
from jax.experimental.pallas import tpu as pltpu
from jax.experimental.pallas import tpu_sc as plsc

_ELEC_ROWS = np.array(
    [
        [0, 0, 0, 0, 0, 0, 0, 0, 0, 0, 0, 0, 0, 0, 0, 0],
        [0, 1, 0, 0, 0, 0, 0, 0, 0, 0, 0, 0, 0, 0, 0, 0],
        [2, 0, 0, 0, 0, 0, 0, 0, 0, 0, 0, 0, 0, 0, 0, 0],
        [2, 0, 0, 1, 0, 0, 0, 0, 0, 0, 0, 0, 0, 0, 0, 0],
        [2, 0, 2, 0, 0, 0, 0, 0, 0, 0, 0, 0, 0, 0, 0, 0],
        [2, 0, 2, 0, 0, 1, 0, 0, 0, 0, 0, 0, 0, 0, 0, 0],
        [2, 0, 2, 0, 0, 2, 0, 0, 0, 0, 0, 0, 0, 0, 0, 0],
        [2, 0, 2, 0, 0, 3, 0, 0, 0, 0, 0, 0, 0, 0, 0, 0],
        [2, 0, 2, 0, 2, 2, 0, 0, 0, 0, 0, 0, 0, 0, 0, 0],
        [2, 0, 2, 0, 4, 1, 0, 0, 0, 0, 0, 0, 0, 0, 0, 0],
    ],
    dtype=np.float32,
)
_ELEC_NORM = _ELEC_ROWS / _ELEC_ROWS.max()
_VPAD = 16
_ELEC_PAD = np.zeros((_VPAD, 16), dtype=np.float32)
_ELEC_PAD[:10] = _ELEC_NORM


def _fused_table_body(elec_ref, nuc_ref, elec_w_ref, ls_w_ref, ls_b_ref, out_ref):
    elec_emb = jnp.dot(elec_ref[...], elec_w_ref[...],
                       preferred_element_type=jnp.float32)
    h = nuc_ref[...] + elec_emb
    h = jnp.dot(h, ls_w_ref[...], preferred_element_type=jnp.float32)
    h = h + ls_b_ref[...]
    out_ref[...] = h * jax.nn.sigmoid(h)


def _compute_fused_table(nuclare_table, elec_W, ls_W, ls_b):
    vocab, num_features = nuclare_table.shape
    nuc_pad = jnp.zeros((_VPAD, num_features), jnp.float32).at[:vocab].set(nuclare_table)
    elec_pad = jnp.asarray(_ELEC_PAD)
    return pl.pallas_call(
        _fused_table_body,
        out_shape=jax.ShapeDtypeStruct((_VPAD, num_features), jnp.float32),
    )(elec_pad, nuc_pad, elec_W, ls_W, ls_b.reshape(1, num_features))


_NBUF = 5


def _make_sc_gather(B, D, nc, ns, chunk=128):
    nw = nc * ns
    b_per_w = B // nw
    n_chunks = b_per_w // chunk
    n_groups = n_chunks // _NBUF
    assert n_chunks % _NBUF == 0
    mesh = plsc.VectorSubcoreMesh(core_axis_name="c", subcore_axis_name="s")

    @functools.partial(
        pl.kernel,
        mesh=mesh,
        out_type=jax.ShapeDtypeStruct((B, D), jnp.float32),
        scratch_types=(
            [pltpu.VMEM((n_chunks, chunk), jnp.int32)]
            + [pltpu.VMEM((chunk, D), jnp.float32) for _ in range(_NBUF)]
            + [pltpu.SemaphoreType.DMA for _ in range(2 * _NBUF)]
        ),
    )
    def gather_kernel(z_hbm, table_hbm, out_hbm, idx_all, *bufs_and_sems):
        rows = bufs_and_sems[:_NBUF]
        gsems = bufs_and_sems[_NBUF:2 * _NBUF]
        wsems = bufs_and_sems[2 * _NBUF:]
        wid = lax.axis_index("s") * nc + lax.axis_index("c")
        chunk0 = wid * n_chunks
        tbl = table_hbm.at[wid]

        pltpu.sync_copy(z_hbm.at[pl.ds(chunk0, n_chunks)], idx_all)

        def gather_wait(b):
            pltpu.make_async_copy(
                tbl.at[idx_all.at[0]], rows[b], gsems[b]).wait()

        def write_wait(b):
            pltpu.make_async_copy(
                rows[b], out_hbm.at[pl.ds(0, chunk)], wsems[b]).wait()

        for b in range(_NBUF):
            pltpu.async_copy(tbl.at[idx_all.at[b]], rows[b], gsems[b])

        def group(g, carry):
            j0 = g * _NBUF
            for b in range(_NBUF):
                gather_wait(b)
                pltpu.async_copy(
                    rows[b],
                    out_hbm.at[pl.ds((chunk0 + j0 + b) * chunk, chunk)],
                    wsems[b])
            for b in range(_NBUF):
                jn = j0 + b + _NBUF

                @pl.when(jn < n_chunks)
                def _():
                    write_wait(b)
                    pltpu.async_copy(tbl.at[idx_all.at[jn]], rows[b], gsems[b])

            return carry

        lax.fori_loop(0, n_groups, group, 0)

        for b in range(_NBUF):
            write_wait(b)

    return gather_kernel


def kernel(z, nuclare_table, elec_W, ls_W, ls_b):
    Bz, L = z.shape
    num_features = nuclare_table.shape[1]
    B = Bz * L

    fused = _compute_fused_table(nuclare_table, elec_W, ls_W, ls_b)

    info = plsc.get_sparse_core_info()
    nw = info.num_cores * info.num_subcores
    gather_kernel = _make_sc_gather(B, num_features, info.num_cores,
                                    info.num_subcores)
    z2d = z.reshape(B // 128, 128).astype(jnp.int32)
    table_rep = jnp.broadcast_to(fused[None], (nw, _VPAD, num_features))
    out = gather_kernel(z2d, table_rep)
    return out.reshape(Bz, L, num_features)


# speedup vs baseline: 30.9076x; 3.7135x over previous
import functools

import numpy as np
import jax
import jax.numpy as jnp
from jax import lax
from jax.experimental import pallas as pl
from jax.experimental.pallas import tpu as pltpu
from jax.experimental.pallas import tpu_sc as plsc

_ELEC_ROWS = np.array(
    [
        [0, 0, 0, 0, 0, 0, 0, 0, 0, 0, 0, 0, 0, 0, 0, 0],
        [0, 1, 0, 0, 0, 0, 0, 0, 0, 0, 0, 0, 0, 0, 0, 0],
        [2, 0, 0, 0, 0, 0, 0, 0, 0, 0, 0, 0, 0, 0, 0, 0],
        [2, 0, 0, 1, 0, 0, 0, 0, 0, 0, 0, 0, 0, 0, 0, 0],
        [2, 0, 2, 0, 0, 0, 0, 0, 0, 0, 0, 0, 0, 0, 0, 0],
        [2, 0, 2, 0, 0, 1, 0, 0, 0, 0, 0, 0, 0, 0, 0, 0],
        [2, 0, 2, 0, 0, 2, 0, 0, 0, 0, 0, 0, 0, 0, 0, 0],
        [2, 0, 2, 0, 0, 3, 0, 0, 0, 0, 0, 0, 0, 0, 0, 0],
        [2, 0, 2, 0, 2, 2, 0, 0, 0, 0, 0, 0, 0, 0, 0, 0],
        [2, 0, 2, 0, 4, 1, 0, 0, 0, 0, 0, 0, 0, 0, 0, 0],
    ],
    dtype=np.float32,
)
_ELEC_NORM = _ELEC_ROWS / _ELEC_ROWS.max()
_VPAD = 16
_ELEC_PAD = np.zeros((_VPAD, 16), dtype=np.float32)
_ELEC_PAD[:10] = _ELEC_NORM


def _fused_table_body(elec_ref, nuc_ref, elec_w_ref, ls_w_ref, ls_b_ref, out_ref):
    elec_emb = jnp.dot(elec_ref[...], elec_w_ref[...],
                       preferred_element_type=jnp.float32)
    h = nuc_ref[...] + elec_emb
    h = jnp.dot(h, ls_w_ref[...], preferred_element_type=jnp.float32)
    h = h + ls_b_ref[...]
    out_ref[...] = h * jax.nn.sigmoid(h)


def _compute_fused_table(nuclare_table, elec_W, ls_W, ls_b):
    vocab, num_features = nuclare_table.shape
    nuc_pad = jnp.zeros((_VPAD, num_features), jnp.float32).at[:vocab].set(nuclare_table)
    elec_pad = jnp.asarray(_ELEC_PAD)
    return pl.pallas_call(
        _fused_table_body,
        out_shape=jax.ShapeDtypeStruct((_VPAD, num_features), jnp.float32),
    )(elec_pad, nuc_pad, elec_W, ls_W, ls_b.reshape(1, num_features))


_NBUF = 5


def _make_sc_gather(B, D, nc, ns, chunk=128):
    nw = nc * ns
    b_per_w = B // nw
    n_chunks = b_per_w // chunk
    n_groups = n_chunks // _NBUF
    assert n_chunks % _NBUF == 0
    mesh = plsc.VectorSubcoreMesh(core_axis_name="c", subcore_axis_name="s")

    @functools.partial(
        pl.kernel,
        mesh=mesh,
        out_type=jax.ShapeDtypeStruct((B, D), jnp.float32),
        scratch_types=(
            [pltpu.VMEM((n_chunks, chunk), jnp.int32)]
            + [pltpu.VMEM((chunk, D), jnp.float32) for _ in range(_NBUF)]
            + [pltpu.VMEM_SHARED((ns, _VPAD, D), jnp.float32)]
            + [pltpu.SemaphoreType.DMA for _ in range(2 * _NBUF)]
        ),
    )
    def gather_kernel(z_hbm, table_hbm, out_hbm, idx_all, *bufs_and_sems):
        rows = bufs_and_sems[:_NBUF]
        spm = bufs_and_sems[_NBUF]
        gsems = bufs_and_sems[_NBUF + 1:2 * _NBUF + 1]
        wsems = bufs_and_sems[2 * _NBUF + 1:]
        sid = lax.axis_index("s")
        wid = sid * nc + lax.axis_index("c")
        chunk0 = wid * n_chunks
        pltpu.sync_copy(table_hbm.at[wid], rows[0].at[pl.ds(0, _VPAD)])
        pltpu.sync_copy(rows[0].at[pl.ds(0, _VPAD)], spm.at[sid])
        tbl = spm.at[sid]

        pltpu.sync_copy(z_hbm.at[pl.ds(chunk0, n_chunks)], idx_all)

        def gather_wait(b):
            pltpu.make_async_copy(
                tbl.at[idx_all.at[0]], rows[b], gsems[b]).wait()

        def write_wait(b):
            pltpu.make_async_copy(
                rows[b], out_hbm.at[pl.ds(0, chunk)], wsems[b]).wait()

        for b in range(_NBUF):
            pltpu.async_copy(tbl.at[idx_all.at[b]], rows[b], gsems[b])

        def group(g, carry):
            j0 = g * _NBUF
            for b in range(_NBUF):
                gather_wait(b)
                pltpu.async_copy(
                    rows[b],
                    out_hbm.at[pl.ds((chunk0 + j0 + b) * chunk, chunk)],
                    wsems[b])
            for b in range(_NBUF):
                jn = j0 + b + _NBUF

                @pl.when(jn < n_chunks)
                def _():
                    write_wait(b)
                    pltpu.async_copy(tbl.at[idx_all.at[jn]], rows[b], gsems[b])

            return carry

        lax.fori_loop(0, n_groups, group, 0)

        for b in range(_NBUF):
            write_wait(b)

    return gather_kernel


def kernel(z, nuclare_table, elec_W, ls_W, ls_b):
    Bz, L = z.shape
    num_features = nuclare_table.shape[1]
    B = Bz * L

    fused = _compute_fused_table(nuclare_table, elec_W, ls_W, ls_b)

    info = plsc.get_sparse_core_info()
    nw = info.num_cores * info.num_subcores
    gather_kernel = _make_sc_gather(B, num_features, info.num_cores,
                                    info.num_subcores)
    z2d = z.reshape(B // 128, 128).astype(jnp.int32)
    table_rep = jnp.broadcast_to(fused[None], (nw, _VPAD, num_features))
    out = gather_kernel(z2d, table_rep)
    return out.reshape(Bz, L, num_features)


# speedup vs baseline: 30.9517x; 1.0014x over previous
import functools

import numpy as np
import jax
import jax.numpy as jnp
from jax import lax
from jax.experimental import pallas as pl
from jax.experimental.pallas import tpu as pltpu
from jax.experimental.pallas import tpu_sc as plsc

_ELEC_ROWS = np.array(
    [
        [0, 0, 0, 0, 0, 0, 0, 0, 0, 0, 0, 0, 0, 0, 0, 0],
        [0, 1, 0, 0, 0, 0, 0, 0, 0, 0, 0, 0, 0, 0, 0, 0],
        [2, 0, 0, 0, 0, 0, 0, 0, 0, 0, 0, 0, 0, 0, 0, 0],
        [2, 0, 0, 1, 0, 0, 0, 0, 0, 0, 0, 0, 0, 0, 0, 0],
        [2, 0, 2, 0, 0, 0, 0, 0, 0, 0, 0, 0, 0, 0, 0, 0],
        [2, 0, 2, 0, 0, 1, 0, 0, 0, 0, 0, 0, 0, 0, 0, 0],
        [2, 0, 2, 0, 0, 2, 0, 0, 0, 0, 0, 0, 0, 0, 0, 0],
        [2, 0, 2, 0, 0, 3, 0, 0, 0, 0, 0, 0, 0, 0, 0, 0],
        [2, 0, 2, 0, 2, 2, 0, 0, 0, 0, 0, 0, 0, 0, 0, 0],
        [2, 0, 2, 0, 4, 1, 0, 0, 0, 0, 0, 0, 0, 0, 0, 0],
    ],
    dtype=np.float32,
)
_ELEC_NORM = _ELEC_ROWS / _ELEC_ROWS.max()
_VPAD = 16
_ELEC_PAD = np.zeros((_VPAD, 16), dtype=np.float32)
_ELEC_PAD[:10] = _ELEC_NORM


def _fused_table_body(elec_ref, nuc_ref, elec_w_ref, ls_w_ref, ls_b_ref, out_ref):
    elec_emb = jnp.dot(elec_ref[...], elec_w_ref[...],
                       preferred_element_type=jnp.float32)
    h = nuc_ref[...] + elec_emb
    h = jnp.dot(h, ls_w_ref[...], preferred_element_type=jnp.float32)
    h = h + ls_b_ref[...]
    out_ref[...] = h * jax.nn.sigmoid(h)


def _compute_fused_table(nuclare_table, elec_W, ls_W, ls_b):
    vocab, num_features = nuclare_table.shape
    nuc_pad = jnp.zeros((_VPAD, num_features), jnp.float32).at[:vocab].set(nuclare_table)
    elec_pad = jnp.asarray(_ELEC_PAD)
    return pl.pallas_call(
        _fused_table_body,
        out_shape=jax.ShapeDtypeStruct((_VPAD, num_features), jnp.float32),
    )(elec_pad, nuc_pad, elec_W, ls_W, ls_b.reshape(1, num_features))


_NBUF = 5


def _make_sc_gather(B, D, nc, ns, chunk=128):
    nw = nc * ns
    b_per_w = B // nw
    n_chunks = b_per_w // chunk
    n_groups = n_chunks // _NBUF
    assert n_chunks % _NBUF == 0
    mesh = plsc.VectorSubcoreMesh(core_axis_name="c", subcore_axis_name="s")

    @functools.partial(
        pl.kernel,
        mesh=mesh,
        out_type=jax.ShapeDtypeStruct((B, D), jnp.float32),
        scratch_types=(
            [pltpu.VMEM((n_chunks, chunk), jnp.int32)]
            + [pltpu.VMEM((chunk, D), jnp.float32) for _ in range(_NBUF)]
            + [pltpu.VMEM_SHARED((ns, _VPAD, D), jnp.float32)]
            + [pltpu.SemaphoreType.DMA for _ in range(2 * _NBUF)]
        ),
    )
    def gather_kernel(z_hbm, table_hbm, out_hbm, idx_all, *bufs_and_sems):
        rows = bufs_and_sems[:_NBUF]
        spm = bufs_and_sems[_NBUF]
        gsems = bufs_and_sems[_NBUF + 1:2 * _NBUF + 1]
        wsems = bufs_and_sems[2 * _NBUF + 1:]
        sid = lax.axis_index("s")
        wid = sid * nc + lax.axis_index("c")
        chunk0 = wid * n_chunks
        pltpu.sync_copy(table_hbm, rows[0].at[pl.ds(0, _VPAD)])
        pltpu.sync_copy(rows[0].at[pl.ds(0, _VPAD)], spm.at[sid])
        tbl = spm.at[sid]

        pltpu.sync_copy(z_hbm.at[pl.ds(chunk0, n_chunks)], idx_all)

        def gather_wait(b):
            pltpu.make_async_copy(
                tbl.at[idx_all.at[0]], rows[b], gsems[b]).wait()

        def write_wait(b):
            pltpu.make_async_copy(
                rows[b], out_hbm.at[pl.ds(0, chunk)], wsems[b]).wait()

        for b in range(_NBUF):
            pltpu.async_copy(tbl.at[idx_all.at[b]], rows[b], gsems[b])

        def group(g, carry):
            j0 = g * _NBUF
            for b in range(_NBUF):
                gather_wait(b)
                pltpu.async_copy(
                    rows[b],
                    out_hbm.at[pl.ds((chunk0 + j0 + b) * chunk, chunk)],
                    wsems[b])
            for b in range(_NBUF):
                jn = j0 + b + _NBUF

                @pl.when(jn < n_chunks)
                def _():
                    write_wait(b)
                    pltpu.async_copy(tbl.at[idx_all.at[jn]], rows[b], gsems[b])

            return carry

        lax.fori_loop(0, n_groups, group, 0)

        for b in range(_NBUF):
            write_wait(b)

    return gather_kernel


def kernel(z, nuclare_table, elec_W, ls_W, ls_b):
    Bz, L = z.shape
    num_features = nuclare_table.shape[1]
    B = Bz * L

    fused = _compute_fused_table(nuclare_table, elec_W, ls_W, ls_b)

    info = plsc.get_sparse_core_info()
    gather_kernel = _make_sc_gather(B, num_features, info.num_cores,
                                    info.num_subcores)
    z2d = z.reshape(B // 128, 128).astype(jnp.int32)
    out = gather_kernel(z2d, fused)
    return out.reshape(Bz, L, num_features)
